# Initial kernel scaffold; baseline (speedup 1.0000x reference)
import jax
import jax.numpy as jnp
from jax.experimental import pallas as pl


def kernel(edge_index, nfeats, efeats, W1, b1, W2, b2, Wp, bp):
    raise NotImplementedError("write your pallas kernel here")


# speedup vs baseline: 3.2159x; 3.2159x over previous
import functools

import jax
import jax.numpy as jnp
from jax import lax
from jax.experimental import pallas as pl
from jax.experimental.pallas import tpu as pltpu
from jax.experimental.pallas import tpu_sc as plsc

NC, NS = 2, 16
NW = NC * NS


def _sc_scatter(dst2d, e2, n_pad):
    R = dst2d.shape[0]
    rpt = n_pad // NS
    mesh = plsc.VectorSubcoreMesh(core_axis_name="c", subcore_axis_name="s",
                                  num_cores=NC, num_subcores=NS)

    @functools.partial(
        pl.kernel,
        out_type=(jax.ShapeDtypeStruct((NC, n_pad, 16), jnp.float32),
                  jax.ShapeDtypeStruct((NW, n_pad), jnp.float32)),
        mesh=mesh,
        scratch_types=[
            pltpu.VMEM((128,), jnp.int32),
            pltpu.VMEM((128, 16), jnp.float32),
            pltpu.VMEM((rpt, 16), jnp.float32),
            pltpu.VMEM((n_pad,), jnp.float32),
            pltpu.VMEM_SHARED((n_pad, 16), jnp.float32),
        ],
        compiler_params=pltpu.CompilerParams(needs_layout_passes=False,
                                             use_tc_tiling_on_sc=False),
    )
    def k(dst_hbm, e2_hbm, summ_out, deg_out, idx_v, rows_v, zbuf, hist, acc):
        c = lax.axis_index("c")
        s = lax.axis_index("s")
        wid = s * NC + c
        z16 = jnp.zeros((16,), jnp.float32)
        ones = jnp.ones((16,), jnp.float32)

        def zbody(i, _):
            zbuf[i, :] = z16
            hist[pl.ds(i * 16, 16)] = z16
            return ()
        lax.fori_loop(0, rpt, zbody, ())
        pltpu.sync_copy(zbuf, acc.at[pl.ds(s * rpt, rpt)])
        plsc.subcore_barrier()

        n_i = (R - wid + NW - 1) // NW

        def body(i, _):
            r = wid + i * NW
            pltpu.sync_copy(dst_hbm.at[r], idx_v)
            pltpu.sync_copy(e2_hbm.at[pl.ds(r * 128, 128)], rows_v)
            pltpu.sync_copy(rows_v, acc.at[idx_v], add=True)
            for g in range(8):
                ii = idx_v[pl.ds(g * 16, 16)]
                plsc.addupdate_scatter(hist, [ii], ones)
            return ()
        lax.fori_loop(0, n_i, body, ())

        plsc.subcore_barrier()
        pltpu.sync_copy(acc.at[pl.ds(s * rpt, rpt)], zbuf)
        pltpu.sync_copy(zbuf, summ_out.at[c, pl.ds(s * rpt, rpt)])
        pltpu.sync_copy(hist, deg_out.at[wid])

    return k(dst2d, e2)


def _sc_deg_only(dst2d, n_pad):
    R = dst2d.shape[0]
    mesh = plsc.VectorSubcoreMesh(core_axis_name="c", subcore_axis_name="s",
                                  num_cores=NC, num_subcores=NS)

    @functools.partial(
        pl.kernel,
        out_type=jax.ShapeDtypeStruct((NW, n_pad), jnp.float32),
        mesh=mesh,
        scratch_types=[
            pltpu.VMEM((128,), jnp.int32),
            pltpu.VMEM((n_pad,), jnp.float32),
        ],
        compiler_params=pltpu.CompilerParams(needs_layout_passes=False),
    )
    def k(dst_hbm, deg_out, idx_v, hist):
        c = lax.axis_index("c")
        s = lax.axis_index("s")
        wid = s * NC + c
        z16 = jnp.zeros((16,), jnp.float32)
        ones = jnp.ones((16,), jnp.float32)

        def zbody(i, _):
            hist[pl.ds(i * 16, 16)] = z16
            return ()
        lax.fori_loop(0, n_pad // 16, zbody, ())

        n_i = (R - wid + NW - 1) // NW

        def body(i, _):
            r = wid + i * NW
            pltpu.sync_copy(dst_hbm.at[r], idx_v)
            for g in range(8):
                ii = idx_v[pl.ds(g * 16, 16)]
                plsc.addupdate_scatter(hist, [ii], ones)
            return ()
        lax.fori_loop(0, n_i, body, ())

        pltpu.sync_copy(hist, deg_out.at[wid])

    return k(dst2d)


def _tc_node(summ, dh, nf, W1aT, W1bT, b1, W2aT, W2bT, b2, Wsd):
    Nn = nf.shape[0]
    n_pad = summ.shape[1]
    BN = 2048
    P = lax.Precision.HIGHEST

    def body(summ_ref, dh_ref, nf_ref, w1a, w1b, b1r, w2a, w2b, b2r, wsd,
             out_ref):
        p = summ_ref[...]
        sm = p[0] + p[1]
        deg = jnp.sum(dh_ref[...], axis=0)
        hn = sm / jnp.maximum(deg, 1.0)[:, None]
        x1 = (jnp.dot(nf_ref[...], w1a[...], precision=P)
              + jnp.dot(hn, w1b[...], precision=P) + b1r[...])
        h1 = jnp.maximum(x1, 0.0)
        x2 = (jnp.dot(h1, w2a[...], precision=P)
              + jnp.dot(hn, w2b[...], precision=P) + b2r[...])
        h2 = jnp.maximum(x2, 0.0)
        out_ref[...] = jnp.dot(h2, wsd[...], precision=P)

    return pl.pallas_call(
        body,
        grid=(n_pad // BN,),
        in_specs=[
            pl.BlockSpec((2, BN, 16), lambda i: (0, i, 0)),
            pl.BlockSpec((NW, BN), lambda i: (0, i)),
            pl.BlockSpec((BN, 128), lambda i: (i, 0)),
            pl.BlockSpec((128, 128), lambda i: (0, 0)),
            pl.BlockSpec((16, 128), lambda i: (0, 0)),
            pl.BlockSpec((1, 128), lambda i: (0, 0)),
            pl.BlockSpec((128, 128), lambda i: (0, 0)),
            pl.BlockSpec((16, 128), lambda i: (0, 0)),
            pl.BlockSpec((1, 128), lambda i: (0, 0)),
            pl.BlockSpec((128, 4), lambda i: (0, 0)),
        ],
        out_specs=pl.BlockSpec((BN, 4), lambda i: (i, 0)),
        out_shape=jax.ShapeDtypeStruct((Nn, 4), jnp.float32),
    )(summ, dh, nf, W1aT, W1bT, b1.reshape(1, 128), W2aT, W2bT,
      b2.reshape(1, 128), Wsd)


def _tc_eterm(e2, WeT, bp):
    E = e2.shape[0]
    BE = 8000

    def body(e_ref, w_ref, b_ref, o_ref):
        o_ref[...] = (jnp.dot(e_ref[...], w_ref[...],
                              precision=lax.Precision.HIGHEST) + b_ref[...])

    return pl.pallas_call(
        body,
        grid=(E // BE,),
        in_specs=[
            pl.BlockSpec((BE, 16), lambda i: (i, 0)),
            pl.BlockSpec((16, 2), lambda i: (0, 0)),
            pl.BlockSpec((1, 2), lambda i: (0, 0)),
        ],
        out_specs=pl.BlockSpec((BE, 2), lambda i: (i, 0)),
        out_shape=jax.ShapeDtypeStruct((E, 2), jnp.float32),
    )(e2, WeT, bp.reshape(1, 2))


def _sc_gather(src2d, dst2d, stab_flat, et_flat):
    R = src2d.shape[0]
    T = stab_flat.shape[0]
    mesh = plsc.VectorSubcoreMesh(core_axis_name="c", subcore_axis_name="s",
                                  num_cores=NC, num_subcores=NS)

    @functools.partial(
        pl.kernel,
        out_type=jax.ShapeDtypeStruct((R * 256,), jnp.float32),
        mesh=mesh,
        scratch_types=[
            pltpu.VMEM((T,), jnp.float32),
            pltpu.VMEM((128,), jnp.int32),
            pltpu.VMEM((128,), jnp.int32),
            pltpu.VMEM((256,), jnp.float32),
            pltpu.VMEM((256,), jnp.float32),
        ],
        compiler_params=pltpu.CompilerParams(needs_layout_passes=False),
    )
    def k(src_hbm, dst_hbm, tab_hbm, et_hbm, out_hbm, tab_v, sv, dv, ev, ov):
        c = lax.axis_index("c")
        s = lax.axis_index("s")
        wid = s * NC + c
        pltpu.sync_copy(tab_hbm, tab_v)
        i2 = lax.iota(jnp.int32, 16) * 2
        n_i = (R - wid + NW - 1) // NW

        def body(i, _):
            r = wid + i * NW
            pltpu.sync_copy(src_hbm.at[r], sv)
            pltpu.sync_copy(dst_hbm.at[r], dv)
            pltpu.sync_copy(et_hbm.at[pl.ds(r * 256, 256)], ev)
            for g in range(8):
                s4 = sv[pl.ds(g * 16, 16)] * 4
                d4 = dv[pl.ds(g * 16, 16)] * 4
                a0 = plsc.load_gather(tab_v, [s4])
                a1 = plsc.load_gather(tab_v, [s4 + 1])
                b0 = plsc.load_gather(tab_v, [d4 + 2])
                b1g = plsc.load_gather(tab_v, [d4 + 3])
                e0 = plsc.load_gather(ev, [i2 + g * 32])
                e1 = plsc.load_gather(ev, [i2 + (g * 32 + 1)])
                plsc.store_scatter(ov, [i2 + g * 32], a0 + b0 + e0)
                plsc.store_scatter(ov, [i2 + (g * 32 + 1)], a1 + b1g + e1)
            pltpu.sync_copy(ov, out_hbm.at[pl.ds(r * 256, 256)])
            return ()
        lax.fori_loop(0, n_i, body, ())

    return k(src2d, dst2d, stab_flat, et_flat)


def kernel(edge_index, nfeats, efeats, W1, b1, W2, b2, Wp, bp):
    E = edge_index.shape[1]
    Nn = nfeats.shape[0]
    D = nfeats.shape[-1]
    ED = efeats.shape[-1]
    src = edge_index[0]
    dst = edge_index[1]
    e2 = efeats.reshape(E, ED)
    n_pad = ((Nn + NS * 16 - 1) // (NS * 16)) * (NS * 16)
    src2d = src.reshape(E // 128, 128)
    dst2d = dst.reshape(E // 128, 128)

    summ, dh = _sc_scatter(dst2d, e2, n_pad)

    W1aT = W1[:, :D].T
    W1bT = W1[:, D:].T
    W2aT = W2[:, :D].T
    W2bT = W2[:, D:].T
    Wsd = jnp.concatenate([Wp[:, :D].T, Wp[:, D:2 * D].T], axis=1)
    WeT = Wp[:, 2 * D:].T

    nf = nfeats.reshape(Nn, D)
    stab = _tc_node(summ, dh, nf,
                    W1aT, W1bT, b1, W2aT, W2bT, b2, Wsd)
    et = _tc_eterm(e2, WeT, bp)
    score_flat = _sc_gather(src2d, dst2d, stab.reshape(-1), et.reshape(-1))
    return score_flat.reshape(E, 2)
